# Initial kernel scaffold; baseline (speedup 1.0000x reference)
import jax
import jax.numpy as jnp
from jax.experimental import pallas as pl


def kernel(in_feats, weight, bias, pairs_in, pairs_out, in_coors):
    raise NotImplementedError("write your pallas kernel here")


# speedup vs baseline: 13.1372x; 13.1372x over previous
import functools

import jax
import jax.numpy as jnp
from jax import lax
from jax.experimental import pallas as pl
from jax.experimental.pallas import tpu as pltpu
from jax.experimental.pallas import tpu_sc as plsc

N = 50000
C = 128
K = 27
NP = 50176
BATCH = 128
NC, NS = 2, 16
NW = NC * NS
CH = 4
R = 12544
RPAD = R + 128
ZR = RPAD // NS
BM = 512
BD = 2000

_mesh = plsc.VectorSubcoreMesh(
    core_axis_name="c", subcore_axis_name="s", num_cores=NC, num_subcores=NS)


def _sload(tab_v, idx):
    row = (idx // 16) * 16
    lane = idx % 16
    v = tab_v[pl.ds(row, 16)]
    li = lax.broadcasted_iota(jnp.int32, (16,), 0)
    return jnp.max(jnp.where(li == lane, v, 0))


@functools.partial(
    pl.kernel,
    out_type=jax.ShapeDtypeStruct((K, NP, C), jnp.float32),
    mesh=_mesh,
    scratch_types=[
        pltpu.VMEM((BATCH,), jnp.int32),
        pltpu.VMEM((BATCH, C), jnp.float32),
        pltpu.VMEM((32,), jnp.int32),
        pltpu.SemaphoreType.DMA,
    ],
    compiler_params=pltpu.CompilerParams(needs_layout_passes=False),
)
def _gather_phase(feats, srcp, cnt_tab, g, idx_v, rows_v, cnt_v, sem):
    cid = lax.axis_index("c")
    sid = lax.axis_index("s")
    wid = sid * NC + cid
    pltpu.sync_copy(cnt_tab, cnt_v)

    def kk_body(kki, carry):
        kk = kki + jnp.where(kki >= 13, 1, 0)
        cnt = _sload(cnt_v, kk)
        nb = (cnt + BATCH - 1) // BATCH
        nt = jnp.maximum(0, (nb - wid + NW - 1) // NW)

        def t_body(t, c2):
            b = wid + t * NW
            p0 = b * BATCH
            pltpu.sync_copy(srcp.at[pl.ds(kk * NP + p0, BATCH)], idx_v)
            for j in range(BATCH // 16):
                sl = pl.ds(j * 16, 16)
                idx_v[sl] = jnp.minimum(idx_v[sl], N - 1)
            pltpu.async_copy(feats.at[idx_v], rows_v, sem).wait()
            pltpu.sync_copy(rows_v, g.at[kk, pl.ds(p0, BATCH)])
            return c2

        return lax.fori_loop(0, nt, t_body, carry)

    lax.fori_loop(0, K - 1, kk_body, 0)


@functools.partial(
    pl.kernel,
    out_type=jax.ShapeDtypeStruct((CH * R, C), jnp.float32),
    mesh=_mesh,
    scratch_types=[
        pltpu.VMEM((BATCH,), jnp.int32),
        pltpu.VMEM((BATCH,), jnp.int32),
        pltpu.VMEM((BATCH, C), jnp.float32),
        pltpu.VMEM((K * 16,), jnp.int32),
        pltpu.VMEM_SHARED((RPAD, C), jnp.float32),
        pltpu.SemaphoreType.DMA,
    ],
    compiler_params=pltpu.CompilerParams(needs_layout_passes=False),
)
def _scatter_phase(p, dstp, rng_tab, zrows, out,
                   idx_v, rel_v, rows_v, rng_v, acc, sem):
    cid = lax.axis_index("c")
    sid = lax.axis_index("s")
    li = lax.broadcasted_iota(jnp.int32, (16,), 0)
    pltpu.sync_copy(rng_tab, rng_v)

    for ci in range(CH // NC):
        chunk = cid + ci * NC
        base = chunk * R
        pltpu.sync_copy(zrows, acc.at[pl.ds(sid * ZR, ZR)])
        plsc.subcore_barrier()

        def kk_body(kki, carry):
            kk = kki + jnp.where(kki >= 13, 1, 0)
            row = rng_v[pl.ds(kk * 16, 16)]
            s = jnp.max(jnp.where(li == chunk, row, 0))
            e = jnp.max(jnp.where(li == chunk + CH, row, 0))
            s8 = (s // 8) * 8
            nb = (e - s8 + BATCH - 1) // BATCH
            nt = jnp.maximum(0, (nb - sid + NS - 1) // NS)

            def t_body(t, c2):
                b = sid + t * NS
                p0 = s8 + b * BATCH
                pltpu.sync_copy(dstp.at[pl.ds(kk * NP + p0, BATCH)], idx_v)
                pltpu.sync_copy(p.at[kk, pl.ds(p0, BATCH)], rows_v)
                for j in range(BATCH // 16):
                    sl = pl.ds(j * 16, 16)
                    pp = p0 + j * 16 + li
                    ok = (pp >= s) & (pp < e)
                    rel_v[sl] = jnp.where(ok, idx_v[sl] - base, R)
                pltpu.sync_copy(rows_v, acc.at[rel_v], add=True)
                return c2

            return lax.fori_loop(0, nt, t_body, carry)

        lax.fori_loop(0, K - 1, kk_body, 0)
        plsc.subcore_barrier()
        WR = R // NS
        pltpu.sync_copy(acc.at[pl.ds(sid * WR, WR)],
                        out.at[pl.ds(base + sid * WR, WR)])
        plsc.subcore_barrier()


def _mm_body(nb_ref, g, w, pout, gbuf, pbuf, sem_in, sem_out):
    kk = pl.program_id(0)
    nb = nb_ref[kk]

    def body(b, carry):
        cin = pltpu.make_async_copy(g.at[kk, pl.ds(b * BM, BM)], gbuf, sem_in)
        cin.start()
        cin.wait()
        pbuf[...] = jnp.dot(gbuf[...], w[0],
                            preferred_element_type=jnp.float32)
        cout = pltpu.make_async_copy(pbuf, pout.at[kk, pl.ds(b * BM, BM)],
                                     sem_out)
        cout.start()
        cout.wait()
        return carry

    lax.fori_loop(0, nb, body, 0)


def _fin_body(acc_b, feats_b, w13_b, bias_b, o_b):
    o_b[...] = jnp.maximum(
        acc_b[...]
        + jnp.dot(feats_b[...], w13_b[...], preferred_element_type=jnp.float32)
        + bias_b[...],
        0.0,
    )


def kernel(in_feats, weight, bias, pairs_in, pairs_out, in_coors):
    pi = pairs_in.astype(jnp.int32)
    po = pairs_out.astype(jnp.int32)
    cnt = jnp.sum(pi < N, axis=1, dtype=jnp.int32)

    cnt_tab = jnp.zeros((32,), jnp.int32).at[:K].set(cnt)
    src_pad = jnp.pad(po, ((0, 0), (0, NP - N)), constant_values=N).reshape(-1)
    dst_pad = jnp.pad(pi, ((0, 0), (0, NP - N)), constant_values=N).reshape(-1)

    bounds_lo = jnp.array([0, R, 2 * R, 3 * R], jnp.int32)
    bounds_hi = jnp.array([R, 2 * R, 3 * R, N], jnp.int32)
    ss_lo = jax.vmap(
        lambda r: jnp.searchsorted(r, bounds_lo).astype(jnp.int32))(pi)
    ss_hi = jax.vmap(
        lambda r: jnp.searchsorted(r, bounds_hi).astype(jnp.int32))(pi)
    rng_tab = jnp.concatenate(
        [ss_lo, ss_hi, jnp.zeros((K, 8), jnp.int32)], axis=1).reshape(-1)

    zrows = jnp.zeros((ZR, C), jnp.float32)
    wsym = weight[::-1]
    nbb = ((cnt + BM - 1) // BM).at[13].set(0).astype(jnp.int32)

    g_arr = _gather_phase(in_feats, src_pad, cnt_tab)

    p_arr = pl.pallas_call(
        _mm_body,
        grid_spec=pltpu.PrefetchScalarGridSpec(
            num_scalar_prefetch=1,
            grid=(K,),
            in_specs=[
                pl.BlockSpec(memory_space=pltpu.MemorySpace.HBM),
                pl.BlockSpec((1, C, C), lambda kk, nb: (kk, 0, 0)),
            ],
            out_specs=pl.BlockSpec(memory_space=pltpu.MemorySpace.HBM),
            scratch_shapes=[
                pltpu.VMEM((BM, C), jnp.float32),
                pltpu.VMEM((BM, C), jnp.float32),
                pltpu.SemaphoreType.DMA,
                pltpu.SemaphoreType.DMA,
            ],
        ),
        out_shape=jax.ShapeDtypeStruct((K, NP, C), jnp.float32),
    )(nbb, g_arr, wsym)

    acc_arr = _scatter_phase(p_arr, dst_pad, rng_tab, zrows)

    out_feats = pl.pallas_call(
        _fin_body,
        grid=(N // BD,),
        in_specs=[
            pl.BlockSpec((BD, C), lambda b: (b, 0)),
            pl.BlockSpec((BD, C), lambda b: (b, 0)),
            pl.BlockSpec((C, C), lambda b: (0, 0)),
            pl.BlockSpec((1, C), lambda b: (0, 0)),
        ],
        out_specs=pl.BlockSpec((BD, C), lambda b: (b, 0)),
        out_shape=jax.ShapeDtypeStruct((N, C), jnp.float32),
    )(acc_arr, in_feats, weight[13], bias.reshape(1, C))

    return (out_feats, in_coors)


# speedup vs baseline: 17.6740x; 1.3453x over previous
import functools

import jax
import jax.numpy as jnp
from jax import lax
from jax.experimental import pallas as pl
from jax.experimental.pallas import tpu as pltpu
from jax.experimental.pallas import tpu_sc as plsc

N = 50000
C = 128
K = 27
NP = 50176
BATCH = 128
NC, NS = 2, 16
NW = NC * NS
CH = 4
R = 12544
RPAD = R + 128
ZR = RPAD // NS
BM = 512
MAXB = K * (NP // BM)
BD = 2000

_mesh = plsc.VectorSubcoreMesh(
    core_axis_name="c", subcore_axis_name="s", num_cores=NC, num_subcores=NS)


def _sload(tab_v, idx):
    row = (idx // 16) * 16
    lane = idx % 16
    v = tab_v[pl.ds(row, 16)]
    li = lax.broadcasted_iota(jnp.int32, (16,), 0)
    return jnp.max(jnp.where(li == lane, v, 0))


@functools.partial(
    pl.kernel,
    out_type=jax.ShapeDtypeStruct((K, NP, C), jnp.float32),
    mesh=_mesh,
    scratch_types=[
        pltpu.VMEM((BATCH,), jnp.int32),
        pltpu.VMEM((BATCH, C), jnp.float32),
        pltpu.VMEM((32,), jnp.int32),
        pltpu.SemaphoreType.DMA,
    ],
    compiler_params=pltpu.CompilerParams(needs_layout_passes=False),
)
def _gather_phase(feats, srcp, cnt_tab, g, idx_v, rows_v, cnt_v, sem):
    cid = lax.axis_index("c")
    sid = lax.axis_index("s")
    wid = sid * NC + cid
    pltpu.sync_copy(cnt_tab, cnt_v)

    def kk_body(kki, carry):
        kk = kki + jnp.where(kki >= 13, 1, 0)
        cnt = _sload(cnt_v, kk)
        nb = (cnt + BATCH - 1) // BATCH
        nt = jnp.maximum(0, (nb - wid + NW - 1) // NW)

        def t_body(t, c2):
            b = wid + t * NW
            p0 = b * BATCH
            pltpu.sync_copy(srcp.at[pl.ds(kk * NP + p0, BATCH)], idx_v)
            for j in range(BATCH // 16):
                sl = pl.ds(j * 16, 16)
                idx_v[sl] = jnp.minimum(idx_v[sl], N - 1)
            pltpu.async_copy(feats.at[idx_v], rows_v, sem).wait()
            pltpu.sync_copy(rows_v, g.at[kk, pl.ds(p0, BATCH)])
            return c2

        return lax.fori_loop(0, nt, t_body, carry)

    lax.fori_loop(0, K - 1, kk_body, 0)


@functools.partial(
    pl.kernel,
    out_type=jax.ShapeDtypeStruct((CH * R, C), jnp.float32),
    mesh=_mesh,
    scratch_types=[
        pltpu.VMEM((BATCH,), jnp.int32),
        pltpu.VMEM((BATCH,), jnp.int32),
        pltpu.VMEM((BATCH, C), jnp.float32),
        pltpu.VMEM((K * 16,), jnp.int32),
        pltpu.VMEM_SHARED((RPAD, C), jnp.float32),
        pltpu.SemaphoreType.DMA,
    ],
    compiler_params=pltpu.CompilerParams(needs_layout_passes=False),
)
def _scatter_phase(p, dstp, rng_tab, zrows, out,
                   idx_v, rel_v, rows_v, rng_v, acc, sem):
    cid = lax.axis_index("c")
    sid = lax.axis_index("s")
    li = lax.broadcasted_iota(jnp.int32, (16,), 0)
    pltpu.sync_copy(rng_tab, rng_v)

    for ci in range(CH // NC):
        chunk = cid + ci * NC
        base = chunk * R
        pltpu.sync_copy(zrows, acc.at[pl.ds(sid * ZR, ZR)])
        plsc.subcore_barrier()

        def kk_body(kki, carry):
            kk = kki + jnp.where(kki >= 13, 1, 0)
            row = rng_v[pl.ds(kk * 16, 16)]
            s = jnp.max(jnp.where(li == chunk, row, 0))
            e = jnp.max(jnp.where(li == chunk + CH, row, 0))
            s8 = (s // 8) * 8
            nb = (e - s8 + BATCH - 1) // BATCH
            nt = jnp.maximum(0, (nb - sid + NS - 1) // NS)

            def t_body(t, c2):
                b = sid + t * NS
                p0 = s8 + b * BATCH
                pltpu.sync_copy(dstp.at[pl.ds(kk * NP + p0, BATCH)], idx_v)
                pltpu.sync_copy(p.at[kk, pl.ds(p0, BATCH)], rows_v)
                for j in range(BATCH // 16):
                    sl = pl.ds(j * 16, 16)
                    pp = p0 + j * 16 + li
                    ok = (pp >= s) & (pp < e)
                    rel_v[sl] = jnp.where(ok, idx_v[sl] - base, R)
                pltpu.sync_copy(rows_v, acc.at[rel_v], add=True)
                return c2

            return lax.fori_loop(0, nt, t_body, carry)

        lax.fori_loop(0, K - 1, kk_body, 0)
        plsc.subcore_barrier()
        WR = R // NS
        pltpu.sync_copy(acc.at[pl.ds(sid * WR, WR)],
                        out.at[pl.ds(base + sid * WR, WR)])
        plsc.subcore_barrier()


def _mm_body(kk_tab, b_tab, g, w, pout):
    pout[...] = jnp.dot(g[0], w[0], preferred_element_type=jnp.float32)[None]


def _fin_body(acc_b, feats_b, w13_b, bias_b, o_b):
    o_b[...] = jnp.maximum(
        acc_b[...]
        + jnp.dot(feats_b[...], w13_b[...], preferred_element_type=jnp.float32)
        + bias_b[...],
        0.0,
    )


def kernel(in_feats, weight, bias, pairs_in, pairs_out, in_coors):
    pi = pairs_in.astype(jnp.int32)
    po = pairs_out.astype(jnp.int32)
    cnt = jnp.sum(pi < N, axis=1, dtype=jnp.int32)

    cnt_tab = jnp.zeros((32,), jnp.int32).at[:K].set(cnt)
    src_pad = jnp.pad(po, ((0, 0), (0, NP - N)), constant_values=N).reshape(-1)
    dst_pad = jnp.pad(pi, ((0, 0), (0, NP - N)), constant_values=N).reshape(-1)

    bounds_lo = jnp.array([0, R, 2 * R, 3 * R], jnp.int32)
    bounds_hi = jnp.array([R, 2 * R, 3 * R, N], jnp.int32)
    ss_lo = jax.vmap(
        lambda r: jnp.searchsorted(r, bounds_lo).astype(jnp.int32))(pi)
    ss_hi = jax.vmap(
        lambda r: jnp.searchsorted(r, bounds_hi).astype(jnp.int32))(pi)
    rng_tab = jnp.concatenate(
        [ss_lo, ss_hi, jnp.zeros((K, 8), jnp.int32)], axis=1).reshape(-1)

    zrows = jnp.zeros((ZR, C), jnp.float32)
    wsym = weight[::-1]
    nbb = ((cnt + BM - 1) // BM).at[13].set(0).astype(jnp.int32)
    nblocks = jnp.sum(nbb)
    offs = jnp.cumsum(nbb) - nbb
    kk_tab = jnp.repeat(jnp.arange(K, dtype=jnp.int32), nbb,
                        total_repeat_length=MAXB)
    b_tab = (jnp.arange(MAXB, dtype=jnp.int32)
             - jnp.repeat(offs.astype(jnp.int32), nbb,
                          total_repeat_length=MAXB))

    g_arr = _gather_phase(in_feats, src_pad, cnt_tab)

    p_arr = pl.pallas_call(
        _mm_body,
        grid_spec=pltpu.PrefetchScalarGridSpec(
            num_scalar_prefetch=2,
            grid=(nblocks,),
            in_specs=[
                pl.BlockSpec((1, BM, C), lambda i, kt, bt: (kt[i], bt[i], 0)),
                pl.BlockSpec((1, C, C), lambda i, kt, bt: (kt[i], 0, 0)),
            ],
            out_specs=pl.BlockSpec((1, BM, C),
                                   lambda i, kt, bt: (kt[i], bt[i], 0)),
        ),
        out_shape=jax.ShapeDtypeStruct((K, NP, C), jnp.float32),
    )(kk_tab, b_tab, g_arr, wsym)

    acc_arr = _scatter_phase(p_arr, dst_pad, rng_tab, zrows)

    out_feats = pl.pallas_call(
        _fin_body,
        grid=(N // BD,),
        in_specs=[
            pl.BlockSpec((BD, C), lambda b: (b, 0)),
            pl.BlockSpec((BD, C), lambda b: (b, 0)),
            pl.BlockSpec((C, C), lambda b: (0, 0)),
            pl.BlockSpec((1, C), lambda b: (0, 0)),
        ],
        out_specs=pl.BlockSpec((BD, C), lambda b: (b, 0)),
        out_shape=jax.ShapeDtypeStruct((N, C), jnp.float32),
    )(acc_arr, in_feats, weight[13], bias.reshape(1, C))

    return (out_feats, in_coors)
